# Initial kernel scaffold; baseline (speedup 1.0000x reference)
import jax
import jax.numpy as jnp
from jax.experimental import pallas as pl


def kernel(x, edge_index, W1_0, Wr_0, br_0, g_0, b_0, W1_1, Wr_1, br_1, g_1, b_1, W1_2, Wr_2, br_2, g_2, b_2, Wf, bf):
    raise NotImplementedError("write your pallas kernel here")


# speedup vs baseline: 5.0550x; 5.0550x over previous
import functools

import jax
import jax.numpy as jnp
from jax import lax
from jax.experimental import pallas as pl
from jax.experimental.pallas import tpu as pltpu
from jax.experimental.pallas import tpu_sc as plsc

N = 10000
E = 320000
D = 128

NC = 2
NS = 16
CHUNK = 80
EDGES_PER_TILE = E // NS
NCHUNK = EDGES_PER_TILE // CHUNK
SEC = 50
NSEC = NCHUNK // SEC
NPAD = 10240
ACC_ROWS_PER_TILE = NPAD // NS
WB = CHUNK

_mesh = plsc.VectorSubcoreMesh(core_axis_name="c", subcore_axis_name="s")


def _dotT(a, b):
    return jax.lax.dot_general(
        a, b, (((1,), (1,)), ((), ())),
        precision=jax.lax.Precision.HIGHEST,
        preferred_element_type=jnp.float32)


@functools.partial(
    pl.kernel,
    mesh=_mesh,
    out_type=jax.ShapeDtypeStruct((2, NPAD, D), jnp.float32),
    scratch_types=[
        pltpu.VMEM((SEC, CHUNK), jnp.int32),
        pltpu.VMEM((SEC, CHUNK), jnp.int32),
        pltpu.VMEM((CHUNK, D), jnp.float32),
        pltpu.VMEM_SHARED((NPAD, D), jnp.float32),
        pltpu.SemaphoreType.DMA,
    ])
def _segsum(xt_hbm, edge_hbm, s_hbm, gidx, sidx, rows, acc, sem):
    c = lax.axis_index("c")
    s = lax.axis_index("s")

    @pl.loop(0, WB)
    def _(r):
        @pl.loop(0, D, step=16)
        def _(j):
            rows[r, pl.ds(j, 16)] = jnp.zeros((16,), jnp.float32)

    base = s * ACC_ROWS_PER_TILE

    @pl.loop(0, ACC_ROWS_PER_TILE // WB)
    def _(k):
        pltpu.sync_copy(rows, acc.at[pl.ds(base + k * WB, WB)])

    plsc.subcore_barrier()

    @pl.loop(0, NSEC)
    def _(k):
        pltpu.sync_copy(edge_hbm.at[c, s, k], gidx)
        pltpu.sync_copy(edge_hbm.at[1 - c, s, k], sidx)

        @pl.loop(0, SEC)
        def _(i):
            pltpu.async_copy(xt_hbm.at[gidx.at[i]], rows, sem).wait()
            pltpu.sync_copy(rows, acc.at[sidx.at[i]], add=True)

    plsc.subcore_barrier()

    @pl.loop(0, ACC_ROWS_PER_TILE // WB)
    def _(k):
        pltpu.sync_copy(acc.at[pl.ds(base + k * WB, WB)], rows)
        pltpu.sync_copy(rows, s_hbm.at[c, pl.ds(base + k * WB, WB)])


@functools.partial(
    pl.kernel,
    mesh=_mesh,
    out_type=jax.ShapeDtypeStruct((2, NPAD, D), jnp.float32),
    scratch_types=[
        pltpu.VMEM((SEC, CHUNK), jnp.int32),
        pltpu.VMEM((CHUNK, D), jnp.float32),
        pltpu.VMEM_SHARED((NPAD, D), jnp.float32),
    ])
def _counts(edge_hbm, c_hbm, sidx, ones, acc):
    c = lax.axis_index("c")
    s = lax.axis_index("s")

    @pl.loop(0, CHUNK)
    def _(r):
        @pl.loop(0, D, step=16)
        def _(j):
            ones[r, pl.ds(j, 16)] = jnp.zeros((16,), jnp.float32)

    base = s * ACC_ROWS_PER_TILE

    @pl.loop(0, ACC_ROWS_PER_TILE // WB)
    def _(k):
        pltpu.sync_copy(ones, acc.at[pl.ds(base + k * WB, WB)])

    @pl.loop(0, CHUNK)
    def _(r):
        @pl.loop(0, D, step=16)
        def _(j):
            ones[r, pl.ds(j, 16)] = jnp.full((16,), 1.0, jnp.float32)

    plsc.subcore_barrier()

    @pl.loop(0, NSEC)
    def _(k):
        pltpu.sync_copy(edge_hbm.at[1 - c, s, k], sidx)

        @pl.loop(0, SEC)
        def _(i):
            pltpu.sync_copy(ones, acc.at[sidx.at[i]], add=True)

    plsc.subcore_barrier()

    @pl.loop(0, ACC_ROWS_PER_TILE // WB)
    def _(k):
        pltpu.sync_copy(acc.at[pl.ds(base + k * WB, WB)], ones)
        pltpu.sync_copy(ones, c_hbm.at[c, pl.ds(base + k * WB, WB)])


BN = 2000


def _mm_body(x_ref, w1_ref, wr_ref, br_ref, xt_ref, yr_ref):
    xb = x_ref[...]
    xt_ref[...] = _dotT(xb, w1_ref[...])
    yr_ref[...] = _dotT(xb, wr_ref[...]) + br_ref[...]


def _mm(x, w1, wr, br):
    return pl.pallas_call(
        _mm_body,
        grid=(N // BN,),
        in_specs=[
            pl.BlockSpec((BN, D), lambda i: (i, 0)),
            pl.BlockSpec((D, D), lambda i: (0, 0)),
            pl.BlockSpec((D, D), lambda i: (0, 0)),
            pl.BlockSpec((1, D), lambda i: (0, 0)),
        ],
        out_specs=[
            pl.BlockSpec((BN, D), lambda i: (i, 0)),
            pl.BlockSpec((BN, D), lambda i: (i, 0)),
        ],
        out_shape=[jax.ShapeDtypeStruct((N, D), jnp.float32)] * 2,
    )(x, w1, wr, br)


def _combine_body(yr_ref, s1_ref, s2_ref, c1_ref, c2_ref, g_ref, b_ref, h_ref):
    r1 = 1.0 / jnp.maximum(c1_ref[...], 1.0)
    r2 = 1.0 / jnp.maximum(c2_ref[...], 1.0)
    t = yr_ref[...] + s1_ref[...] * r1 + s2_ref[...] * r2
    t = jnp.maximum(t, 0.0)
    mu = jnp.mean(t, axis=1, keepdims=True)
    var = jnp.mean((t - mu) * (t - mu), axis=1, keepdims=True)
    h_ref[...] = (t - mu) * jax.lax.rsqrt(var + 1e-5) * g_ref[...] + b_ref[...]


def _combine(yr, s1, s2, c1, c2, g, b):
    return pl.pallas_call(
        _combine_body,
        grid=(N // BN,),
        in_specs=[
            pl.BlockSpec((BN, D), lambda i: (i, 0)),
            pl.BlockSpec((BN, D), lambda i: (i, 0)),
            pl.BlockSpec((BN, D), lambda i: (i, 0)),
            pl.BlockSpec((BN, D), lambda i: (i, 0)),
            pl.BlockSpec((BN, D), lambda i: (i, 0)),
            pl.BlockSpec((1, D), lambda i: (0, 0)),
            pl.BlockSpec((1, D), lambda i: (0, 0)),
        ],
        out_specs=pl.BlockSpec((BN, D), lambda i: (i, 0)),
        out_shape=jax.ShapeDtypeStruct((N, D), jnp.float32),
    )(yr, s1, s2, c1, c2, g, b)


def _final_body(x0_ref, x1_ref, x2_ref, x3_ref, wf_ref, bf_ref, o_ref):
    wf = wf_ref[...]
    acc = _dotT(x0_ref[...], wf[:, 0:D])
    acc += _dotT(x1_ref[...], wf[:, D:2 * D])
    acc += _dotT(x2_ref[...], wf[:, 2 * D:3 * D])
    acc += _dotT(x3_ref[...], wf[:, 3 * D:4 * D])
    o_ref[...] = acc + bf_ref[...]


def _final(x0, x1, x2, x3, wf, bf):
    return pl.pallas_call(
        _final_body,
        grid=(N // BN,),
        in_specs=[
            pl.BlockSpec((BN, D), lambda i: (i, 0)),
            pl.BlockSpec((BN, D), lambda i: (i, 0)),
            pl.BlockSpec((BN, D), lambda i: (i, 0)),
            pl.BlockSpec((BN, D), lambda i: (i, 0)),
            pl.BlockSpec((D, 4 * D), lambda i: (0, 0)),
            pl.BlockSpec((1, D), lambda i: (0, 0)),
        ],
        out_specs=pl.BlockSpec((BN, D), lambda i: (i, 0)),
        out_shape=jax.ShapeDtypeStruct((N, D), jnp.float32),
    )(x0, x1, x2, x3, wf, bf)


def kernel(x, edge_index, W1_0, Wr_0, br_0, g_0, b_0, W1_1, Wr_1, br_1, g_1,
           b_1, W1_2, Wr_2, br_2, g_2, b_2, Wf, bf):
    edge_r = edge_index.reshape(2, NS, NSEC, SEC, CHUNK)
    cnt = _counts(edge_r)
    c1 = cnt[0, :N]
    c2 = cnt[1, :N]

    W1s = jnp.stack([W1_0, W1_1, W1_2])
    Wrs = jnp.stack([Wr_0, Wr_1, Wr_2])
    brs = jnp.stack([br_0, br_1, br_2]).reshape(3, 1, D)
    gs = jnp.stack([g_0, g_1, g_2]).reshape(3, 1, D)
    bs = jnp.stack([b_0, b_1, b_2]).reshape(3, 1, D)

    def step(h, p):
        W1, Wr, br, g, b = p
        xt, yr = _mm(h, W1, Wr, br)
        S = _segsum(xt, edge_r)
        h2 = _combine(yr, S[0, :N], S[1, :N], c1, c2, g, b)
        return h2, h2

    _, hs = jax.lax.scan(step, x, (W1s, Wrs, brs, gs, bs))
    return _final(x, hs[0], hs[1], hs[2], Wf, bf.reshape(1, D))


# speedup vs baseline: 7.7675x; 1.5366x over previous
import functools

import jax
import jax.numpy as jnp
from jax import lax
from jax.experimental import pallas as pl
from jax.experimental.pallas import tpu as pltpu
from jax.experimental.pallas import tpu_sc as plsc

N = 10000
E = 320000
D = 128

NC = 2
NS = 16
CHUNK = 80
EDGES_PER_TILE = E // NS
NCHUNK = EDGES_PER_TILE // CHUNK
SEC = 50
NSEC = NCHUNK // SEC
NPAD = 10240
ACC_ROWS_PER_TILE = NPAD // NS
WB = CHUNK

_mesh = plsc.VectorSubcoreMesh(core_axis_name="c", subcore_axis_name="s")


def _dotT(a, b):
    return jax.lax.dot_general(
        a, b, (((1,), (1,)), ((), ())),
        precision=jax.lax.Precision.HIGHEST,
        preferred_element_type=jnp.float32)


@functools.partial(
    pl.kernel,
    mesh=_mesh,
    out_type=jax.ShapeDtypeStruct((2, NPAD, D), jnp.float32),
    scratch_types=[
        pltpu.VMEM((SEC, CHUNK), jnp.int32),
        pltpu.VMEM((SEC, CHUNK), jnp.int32),
        pltpu.VMEM((CHUNK, D), jnp.float32),
        pltpu.VMEM((CHUNK, D), jnp.float32),
        pltpu.VMEM_SHARED((NPAD, D), jnp.float32),
        pltpu.SemaphoreType.DMA,
        pltpu.SemaphoreType.DMA,
    ])
def _segsum(xt_hbm, edge_hbm, s_hbm, gidx, sidx, rows0, rows1, acc, sem0, sem1):
    c = lax.axis_index("c")
    s = lax.axis_index("s")

    @pl.loop(0, WB)
    def _(r):
        @pl.loop(0, D, step=16)
        def _(j):
            rows0[r, pl.ds(j, 16)] = jnp.zeros((16,), jnp.float32)

    base = s * ACC_ROWS_PER_TILE

    @pl.loop(0, ACC_ROWS_PER_TILE // WB)
    def _(k):
        pltpu.sync_copy(rows0, acc.at[pl.ds(base + k * WB, WB)])

    plsc.subcore_barrier()

    def _g_start(i, buf, sem):
        pltpu.make_async_copy(xt_hbm.at[gidx.at[i]], buf, sem).start()

    def _g_wait(buf, sem):
        pltpu.make_async_copy(xt_hbm.at[gidx.at[0]], buf, sem).wait()

    @pl.loop(0, NSEC)
    def _(k):
        pltpu.sync_copy(edge_hbm.at[c, s, k], gidx)
        pltpu.sync_copy(edge_hbm.at[1 - c, s, k], sidx)

        _g_start(0, rows0, sem0)

        @pl.loop(0, SEC // 2 - 1)
        def _(i2):
            a = 2 * i2
            _g_start(a + 1, rows1, sem1)
            _g_wait(rows0, sem0)
            pltpu.sync_copy(rows0, acc.at[sidx.at[a]], add=True)
            _g_start(a + 2, rows0, sem0)
            _g_wait(rows1, sem1)
            pltpu.sync_copy(rows1, acc.at[sidx.at[a + 1]], add=True)

        _g_start(SEC - 1, rows1, sem1)
        _g_wait(rows0, sem0)
        pltpu.sync_copy(rows0, acc.at[sidx.at[SEC - 2]], add=True)
        _g_wait(rows1, sem1)
        pltpu.sync_copy(rows1, acc.at[sidx.at[SEC - 1]], add=True)

    plsc.subcore_barrier()

    @pl.loop(0, ACC_ROWS_PER_TILE // WB)
    def _(k):
        pltpu.sync_copy(acc.at[pl.ds(base + k * WB, WB)], rows0)
        pltpu.sync_copy(rows0, s_hbm.at[c, pl.ds(base + k * WB, WB)])


@functools.partial(
    pl.kernel,
    mesh=_mesh,
    out_type=jax.ShapeDtypeStruct((2, NPAD, D), jnp.float32),
    scratch_types=[
        pltpu.VMEM((SEC, CHUNK), jnp.int32),
        pltpu.VMEM((CHUNK, D), jnp.float32),
        pltpu.VMEM_SHARED((NPAD, D), jnp.float32),
    ])
def _counts(edge_hbm, c_hbm, sidx, ones, acc):
    c = lax.axis_index("c")
    s = lax.axis_index("s")

    @pl.loop(0, CHUNK)
    def _(r):
        @pl.loop(0, D, step=16)
        def _(j):
            ones[r, pl.ds(j, 16)] = jnp.zeros((16,), jnp.float32)

    base = s * ACC_ROWS_PER_TILE

    @pl.loop(0, ACC_ROWS_PER_TILE // WB)
    def _(k):
        pltpu.sync_copy(ones, acc.at[pl.ds(base + k * WB, WB)])

    @pl.loop(0, CHUNK)
    def _(r):
        @pl.loop(0, D, step=16)
        def _(j):
            ones[r, pl.ds(j, 16)] = jnp.full((16,), 1.0, jnp.float32)

    plsc.subcore_barrier()

    @pl.loop(0, NSEC)
    def _(k):
        pltpu.sync_copy(edge_hbm.at[1 - c, s, k], sidx)

        @pl.loop(0, SEC)
        def _(i):
            pltpu.sync_copy(ones, acc.at[sidx.at[i]], add=True)

    plsc.subcore_barrier()

    @pl.loop(0, ACC_ROWS_PER_TILE // WB)
    def _(k):
        pltpu.sync_copy(acc.at[pl.ds(base + k * WB, WB)], ones)
        pltpu.sync_copy(ones, c_hbm.at[c, pl.ds(base + k * WB, WB)])


BN = 2000


def _mm_body(x_ref, w1_ref, wr_ref, br_ref, xt_ref, yr_ref):
    xb = x_ref[...]
    xt_ref[...] = _dotT(xb, w1_ref[...])
    yr_ref[...] = _dotT(xb, wr_ref[...]) + br_ref[...]


def _mm(x, w1, wr, br):
    return pl.pallas_call(
        _mm_body,
        grid=(N // BN,),
        in_specs=[
            pl.BlockSpec((BN, D), lambda i: (i, 0)),
            pl.BlockSpec((D, D), lambda i: (0, 0)),
            pl.BlockSpec((D, D), lambda i: (0, 0)),
            pl.BlockSpec((1, D), lambda i: (0, 0)),
        ],
        out_specs=[
            pl.BlockSpec((BN, D), lambda i: (i, 0)),
            pl.BlockSpec((BN, D), lambda i: (i, 0)),
        ],
        out_shape=[jax.ShapeDtypeStruct((N, D), jnp.float32)] * 2,
    )(x, w1, wr, br)


def _combine_body(yr_ref, s1_ref, s2_ref, c1_ref, c2_ref, g_ref, b_ref, h_ref):
    r1 = 1.0 / jnp.maximum(c1_ref[...], 1.0)
    r2 = 1.0 / jnp.maximum(c2_ref[...], 1.0)
    t = yr_ref[...] + s1_ref[...] * r1 + s2_ref[...] * r2
    t = jnp.maximum(t, 0.0)
    mu = jnp.mean(t, axis=1, keepdims=True)
    var = jnp.mean((t - mu) * (t - mu), axis=1, keepdims=True)
    h_ref[...] = (t - mu) * jax.lax.rsqrt(var + 1e-5) * g_ref[...] + b_ref[...]


def _combine(yr, s1, s2, c1, c2, g, b):
    return pl.pallas_call(
        _combine_body,
        grid=(N // BN,),
        in_specs=[
            pl.BlockSpec((BN, D), lambda i: (i, 0)),
            pl.BlockSpec((BN, D), lambda i: (i, 0)),
            pl.BlockSpec((BN, D), lambda i: (i, 0)),
            pl.BlockSpec((BN, D), lambda i: (i, 0)),
            pl.BlockSpec((BN, D), lambda i: (i, 0)),
            pl.BlockSpec((1, D), lambda i: (0, 0)),
            pl.BlockSpec((1, D), lambda i: (0, 0)),
        ],
        out_specs=pl.BlockSpec((BN, D), lambda i: (i, 0)),
        out_shape=jax.ShapeDtypeStruct((N, D), jnp.float32),
    )(yr, s1, s2, c1, c2, g, b)


def _final_body(x0_ref, x1_ref, x2_ref, x3_ref, wf_ref, bf_ref, o_ref):
    wf = wf_ref[...]
    acc = _dotT(x0_ref[...], wf[:, 0:D])
    acc += _dotT(x1_ref[...], wf[:, D:2 * D])
    acc += _dotT(x2_ref[...], wf[:, 2 * D:3 * D])
    acc += _dotT(x3_ref[...], wf[:, 3 * D:4 * D])
    o_ref[...] = acc + bf_ref[...]


def _final(x0, x1, x2, x3, wf, bf):
    return pl.pallas_call(
        _final_body,
        grid=(N // BN,),
        in_specs=[
            pl.BlockSpec((BN, D), lambda i: (i, 0)),
            pl.BlockSpec((BN, D), lambda i: (i, 0)),
            pl.BlockSpec((BN, D), lambda i: (i, 0)),
            pl.BlockSpec((BN, D), lambda i: (i, 0)),
            pl.BlockSpec((D, 4 * D), lambda i: (0, 0)),
            pl.BlockSpec((1, D), lambda i: (0, 0)),
        ],
        out_specs=pl.BlockSpec((BN, D), lambda i: (i, 0)),
        out_shape=jax.ShapeDtypeStruct((N, D), jnp.float32),
    )(x0, x1, x2, x3, wf, bf)


def kernel(x, edge_index, W1_0, Wr_0, br_0, g_0, b_0, W1_1, Wr_1, br_1, g_1,
           b_1, W1_2, Wr_2, br_2, g_2, b_2, Wf, bf):
    edge_r = edge_index.reshape(2, NS, NSEC, SEC, CHUNK)
    cnt = _counts(edge_r)
    c1 = cnt[0, :N]
    c2 = cnt[1, :N]

    W1s = jnp.stack([W1_0, W1_1, W1_2])
    Wrs = jnp.stack([Wr_0, Wr_1, Wr_2])
    brs = jnp.stack([br_0, br_1, br_2]).reshape(3, 1, D)
    gs = jnp.stack([g_0, g_1, g_2]).reshape(3, 1, D)
    bs = jnp.stack([b_0, b_1, b_2]).reshape(3, 1, D)

    def step(h, p):
        W1, Wr, br, g, b = p
        xt, yr = _mm(h, W1, Wr, br)
        S = _segsum(xt, edge_r)
        h2 = _combine(yr, S[0, :N], S[1, :N], c1, c2, g, b)
        return h2, h2

    _, hs = jax.lax.scan(step, x, (W1s, Wrs, brs, gs, bs))
    return _final(x, hs[0], hs[1], hs[2], Wf, bf.reshape(1, D))


# speedup vs baseline: 8.0827x; 1.0406x over previous
import functools

import jax
import jax.numpy as jnp
from jax import lax
from jax.experimental import pallas as pl
from jax.experimental.pallas import tpu as pltpu
from jax.experimental.pallas import tpu_sc as plsc

N = 10000
E = 320000
D = 128

NC = 2
NS = 16
CHUNK = 80
EDGES_PER_TILE = E // NS
NCHUNK = EDGES_PER_TILE // CHUNK
SEC = 50
NSEC = NCHUNK // SEC
NPAD = 10240
ACC_ROWS_PER_TILE = NPAD // NS
WB = CHUNK

_mesh = plsc.VectorSubcoreMesh(core_axis_name="c", subcore_axis_name="s")


def _dotT(a, b):
    return jax.lax.dot_general(
        a, b, (((1,), (1,)), ((), ())),
        precision=jax.lax.Precision.HIGHEST,
        preferred_element_type=jnp.float32)


@functools.partial(
    pl.kernel,
    mesh=_mesh,
    out_type=jax.ShapeDtypeStruct((2, NPAD, D), jnp.float32),
    scratch_types=[
        pltpu.VMEM((SEC, CHUNK), jnp.int32),
        pltpu.VMEM((SEC, CHUNK), jnp.int32),
        pltpu.VMEM((CHUNK, D), jnp.float32),
        pltpu.VMEM((CHUNK, D), jnp.float32),
        pltpu.VMEM_SHARED((NPAD, D), jnp.float32),
        pltpu.SemaphoreType.DMA,
        pltpu.SemaphoreType.DMA,
    ])
def _segsum(xt_hbm, edge_hbm, s_hbm, gidx, sidx, rows0, rows1, acc, sem0, sem1):
    c = lax.axis_index("c")
    s = lax.axis_index("s")

    @pl.loop(0, WB)
    def _(r):
        @pl.loop(0, D, step=16)
        def _(j):
            rows0[r, pl.ds(j, 16)] = jnp.zeros((16,), jnp.float32)

    base = s * ACC_ROWS_PER_TILE

    @pl.loop(0, ACC_ROWS_PER_TILE // WB)
    def _(k):
        pltpu.sync_copy(rows0, acc.at[pl.ds(base + k * WB, WB)])

    plsc.subcore_barrier()

    def _g_start(i, buf, sem):
        pltpu.make_async_copy(xt_hbm.at[gidx.at[i]], buf, sem).start()

    def _g_wait(buf, sem):
        pltpu.make_async_copy(xt_hbm.at[gidx.at[0]], buf, sem).wait()

    @pl.loop(0, NSEC)
    def _(k):
        pltpu.sync_copy(edge_hbm.at[c, s, k], gidx)
        pltpu.sync_copy(edge_hbm.at[1 - c, s, k], sidx)

        _g_start(0, rows0, sem0)

        @pl.loop(0, SEC // 2 - 1)
        def _(i2):
            a = 2 * i2
            _g_start(a + 1, rows1, sem1)
            _g_wait(rows0, sem0)
            pltpu.sync_copy(rows0, acc.at[sidx.at[a]], add=True)
            _g_start(a + 2, rows0, sem0)
            _g_wait(rows1, sem1)
            pltpu.sync_copy(rows1, acc.at[sidx.at[a + 1]], add=True)

        _g_start(SEC - 1, rows1, sem1)
        _g_wait(rows0, sem0)
        pltpu.sync_copy(rows0, acc.at[sidx.at[SEC - 2]], add=True)
        _g_wait(rows1, sem1)
        pltpu.sync_copy(rows1, acc.at[sidx.at[SEC - 1]], add=True)

    plsc.subcore_barrier()

    pltpu.sync_copy(acc.at[pl.ds(base, ACC_ROWS_PER_TILE)],
                    s_hbm.at[c, pl.ds(base, ACC_ROWS_PER_TILE)])


@functools.partial(
    pl.kernel,
    mesh=_mesh,
    out_type=jax.ShapeDtypeStruct((2, NPAD, D), jnp.float32),
    scratch_types=[
        pltpu.VMEM((SEC, CHUNK), jnp.int32),
        pltpu.VMEM((CHUNK, D), jnp.float32),
        pltpu.VMEM_SHARED((NPAD, D), jnp.float32),
        pltpu.SemaphoreType.DMA,
    ])
def _counts(edge_hbm, c_hbm, sidx, ones, acc, sem):
    c = lax.axis_index("c")
    s = lax.axis_index("s")

    @pl.loop(0, CHUNK)
    def _(r):
        @pl.loop(0, D, step=16)
        def _(j):
            ones[r, pl.ds(j, 16)] = jnp.zeros((16,), jnp.float32)

    base = s * ACC_ROWS_PER_TILE

    @pl.loop(0, ACC_ROWS_PER_TILE // WB)
    def _(k):
        pltpu.sync_copy(ones, acc.at[pl.ds(base + k * WB, WB)])

    @pl.loop(0, CHUNK)
    def _(r):
        @pl.loop(0, D, step=16)
        def _(j):
            ones[r, pl.ds(j, 16)] = jnp.full((16,), 1.0, jnp.float32)

    plsc.subcore_barrier()

    @pl.loop(0, NSEC)
    def _(k):
        pltpu.sync_copy(edge_hbm.at[1 - c, s, k], sidx)

        @pl.loop(0, SEC // 10)
        def _(q):
            b10 = q * 10
            for j in range(10):
                pltpu.async_copy(ones, acc.at[sidx.at[b10 + j]], sem, add=True)
            for j in range(10):
                pltpu.make_async_copy(ones, acc.at[sidx.at[b10]], sem).wait()

    plsc.subcore_barrier()

    pltpu.sync_copy(acc.at[pl.ds(base, ACC_ROWS_PER_TILE)],
                    c_hbm.at[c, pl.ds(base, ACC_ROWS_PER_TILE)])


BN = 2000


def _tc0_body(x_ref, w1_ref, wr_ref, br_ref, wf0_ref, bf_ref,
              xt_ref, yr_ref, p_ref):
    xb = x_ref[...]
    xt_ref[...] = _dotT(xb, w1_ref[...])
    yr_ref[...] = _dotT(xb, wr_ref[...]) + br_ref[...]
    p_ref[...] = _dotT(xb, wf0_ref[...]) + bf_ref[...]


def _tc0(x, w1, wr, br, wf0, bf):
    return pl.pallas_call(
        _tc0_body,
        grid=(N // BN,),
        in_specs=[
            pl.BlockSpec((BN, D), lambda i: (i, 0)),
            pl.BlockSpec((D, D), lambda i: (0, 0)),
            pl.BlockSpec((D, D), lambda i: (0, 0)),
            pl.BlockSpec((1, D), lambda i: (0, 0)),
            pl.BlockSpec((D, D), lambda i: (0, 0)),
            pl.BlockSpec((1, D), lambda i: (0, 0)),
        ],
        out_specs=[pl.BlockSpec((BN, D), lambda i: (i, 0))] * 3,
        out_shape=[jax.ShapeDtypeStruct((N, D), jnp.float32)] * 3,
    )(x, w1, wr, br, wf0, bf)


def _tcstep_body(yr_ref, s1_ref, s2_ref, c1_ref, c2_ref, p_ref, g_ref, b_ref,
                 wfh_ref, w1_ref, wr_ref, br_ref, xt_ref, yrn_ref, pn_ref):
    r1 = 1.0 / jnp.maximum(c1_ref[...], 1.0)
    r2 = 1.0 / jnp.maximum(c2_ref[...], 1.0)
    t = yr_ref[...] + s1_ref[...] * r1 + s2_ref[...] * r2
    t = jnp.maximum(t, 0.0)
    mu = jnp.mean(t, axis=1, keepdims=True)
    var = jnp.mean((t - mu) * (t - mu), axis=1, keepdims=True)
    h = (t - mu) * jax.lax.rsqrt(var + 1e-5) * g_ref[...] + b_ref[...]
    xt_ref[...] = _dotT(h, w1_ref[...])
    yrn_ref[...] = _dotT(h, wr_ref[...]) + br_ref[...]
    pn_ref[...] = p_ref[...] + _dotT(h, wfh_ref[...])


def _tcstep(yr, s1, s2, c1, c2, p, g, b, wfh, w1n, wrn, brn):
    return pl.pallas_call(
        _tcstep_body,
        grid=(N // BN,),
        in_specs=[
            pl.BlockSpec((BN, D), lambda i: (i, 0)),
            pl.BlockSpec((BN, D), lambda i: (i, 0)),
            pl.BlockSpec((BN, D), lambda i: (i, 0)),
            pl.BlockSpec((BN, D), lambda i: (i, 0)),
            pl.BlockSpec((BN, D), lambda i: (i, 0)),
            pl.BlockSpec((BN, D), lambda i: (i, 0)),
            pl.BlockSpec((1, D), lambda i: (0, 0)),
            pl.BlockSpec((1, D), lambda i: (0, 0)),
            pl.BlockSpec((D, D), lambda i: (0, 0)),
            pl.BlockSpec((D, D), lambda i: (0, 0)),
            pl.BlockSpec((D, D), lambda i: (0, 0)),
            pl.BlockSpec((1, D), lambda i: (0, 0)),
        ],
        out_specs=[pl.BlockSpec((BN, D), lambda i: (i, 0))] * 3,
        out_shape=[jax.ShapeDtypeStruct((N, D), jnp.float32)] * 3,
    )(yr, s1, s2, c1, c2, p, g, b, wfh, w1n, wrn, brn)


def kernel(x, edge_index, W1_0, Wr_0, br_0, g_0, b_0, W1_1, Wr_1, br_1, g_1,
           b_1, W1_2, Wr_2, br_2, g_2, b_2, Wf, bf):
    edge_r = edge_index.reshape(2, NS, NSEC, SEC, CHUNK)
    cnt = _counts(edge_r)
    c1 = cnt[0, :N]
    c2 = cnt[1, :N]

    xt0, yr0, p0 = _tc0(x, W1_0, Wr_0, br_0.reshape(1, D), Wf[:, :D],
                        bf.reshape(1, D))

    gs = jnp.stack([g_0, g_1, g_2]).reshape(3, 1, D)
    bs = jnp.stack([b_0, b_1, b_2]).reshape(3, 1, D)
    wfh = jnp.stack([Wf[:, D:2 * D], Wf[:, 2 * D:3 * D], Wf[:, 3 * D:4 * D]])
    w1n = jnp.stack([W1_1, W1_2, W1_0])
    wrn = jnp.stack([Wr_1, Wr_2, Wr_0])
    brn = jnp.stack([br_1, br_2, br_0]).reshape(3, 1, D)

    def step(carry, pv):
        xt, yr, p = carry
        g, b, wfh_i, w1_i, wr_i, br_i = pv
        S = _segsum(xt, edge_r)
        xt2, yr2, p2 = _tcstep(yr, S[0, :N], S[1, :N], c1, c2, p, g, b,
                               wfh_i, w1_i, wr_i, br_i)
        return (xt2, yr2, p2), None

    (_, _, p3), _ = jax.lax.scan(step, (xt0, yr0, p0),
                                 (gs, bs, wfh, w1n, wrn, brn))
    return p3


# speedup vs baseline: 8.9500x; 1.1073x over previous
import dataclasses
import functools

import jax
import jax.numpy as jnp
from jax import lax
from jax.experimental import pallas as pl
from jax.experimental.pallas import tpu as pltpu
from jax.experimental.pallas import tpu_sc as plsc

N = 10000
E = 320000
D = 128

NC = 2
NS = 16
CHUNK = 80
EDGES_PER_TILE = E // NS
NCHUNK = EDGES_PER_TILE // CHUNK
SEC = 50
NSEC = NCHUNK // SEC
NPAD = 10240
ACC_ROWS_PER_TILE = NPAD // NS
WB = CHUNK

_mesh = plsc.VectorSubcoreMesh(core_axis_name="c", subcore_axis_name="s")

_sc_params = pltpu.CompilerParams()
if "needs_layout_passes" in pltpu.CompilerParams.__dataclass_fields__:
    _sc_params = dataclasses.replace(_sc_params, needs_layout_passes=False)


def _dotT(a, b):
    return jax.lax.dot_general(
        a, b, (((1,), (1,)), ((), ())),
        precision=jax.lax.Precision.HIGHEST,
        preferred_element_type=jnp.float32)


@functools.partial(
    pl.kernel,
    mesh=_mesh,
    out_type=jax.ShapeDtypeStruct((2, NPAD, D), jnp.float32),
    scratch_types=[
        pltpu.VMEM((SEC, CHUNK), jnp.int32),
        pltpu.VMEM((SEC, CHUNK), jnp.int32),
        pltpu.VMEM((CHUNK, D), jnp.float32),
        pltpu.VMEM((CHUNK, D), jnp.float32),
        pltpu.VMEM_SHARED((NPAD, D), jnp.float32),
        pltpu.SemaphoreType.DMA,
        pltpu.SemaphoreType.DMA,
    ])
def _segsum(xt_hbm, edge_hbm, s_hbm, gidx, sidx, rows0, rows1, acc, sem0, sem1):
    c = lax.axis_index("c")
    s = lax.axis_index("s")

    @pl.loop(0, WB)
    def _(r):
        @pl.loop(0, D, step=16)
        def _(j):
            rows0[r, pl.ds(j, 16)] = jnp.zeros((16,), jnp.float32)

    base = s * ACC_ROWS_PER_TILE

    @pl.loop(0, ACC_ROWS_PER_TILE // WB)
    def _(k):
        pltpu.sync_copy(rows0, acc.at[pl.ds(base + k * WB, WB)])

    plsc.subcore_barrier()

    def _g_start(i, buf, sem):
        pltpu.make_async_copy(xt_hbm.at[gidx.at[i]], buf, sem).start()

    def _g_wait(buf, sem):
        pltpu.make_async_copy(xt_hbm.at[gidx.at[0]], buf, sem).wait()

    @pl.loop(0, NSEC)
    def _(k):
        pltpu.sync_copy(edge_hbm.at[c, s, k], gidx)
        pltpu.sync_copy(edge_hbm.at[1 - c, s, k], sidx)

        _g_start(0, rows0, sem0)

        @pl.loop(0, SEC // 2 - 1)
        def _(i2):
            a = 2 * i2
            _g_start(a + 1, rows1, sem1)
            _g_wait(rows0, sem0)
            pltpu.sync_copy(rows0, acc.at[sidx.at[a]], add=True)
            _g_start(a + 2, rows0, sem0)
            _g_wait(rows1, sem1)
            pltpu.sync_copy(rows1, acc.at[sidx.at[a + 1]], add=True)

        _g_start(SEC - 1, rows1, sem1)
        _g_wait(rows0, sem0)
        pltpu.sync_copy(rows0, acc.at[sidx.at[SEC - 2]], add=True)
        _g_wait(rows1, sem1)
        pltpu.sync_copy(rows1, acc.at[sidx.at[SEC - 1]], add=True)

    plsc.subcore_barrier()

    pltpu.sync_copy(acc.at[pl.ds(base, ACC_ROWS_PER_TILE)],
                    s_hbm.at[c, pl.ds(base, ACC_ROWS_PER_TILE)])


RED = ACC_ROWS_PER_TILE // 16


@functools.partial(
    pl.kernel,
    mesh=_mesh,
    out_type=jax.ShapeDtypeStruct((2, NS, ACC_ROWS_PER_TILE // 128, 128),
                                  jnp.float32),
    compiler_params=_sc_params,
    scratch_types=[
        pltpu.VMEM((SEC, CHUNK), jnp.int32),
        pltpu.VMEM((NPAD,), jnp.float32),
        pltpu.VMEM((NS, ACC_ROWS_PER_TILE), jnp.float32),
        pltpu.VMEM((ACC_ROWS_PER_TILE // 128, 128), jnp.float32),
        pltpu.VMEM_SHARED((NS, NPAD), jnp.float32),
    ])
def _counts(edge_hbm, c_hbm, sidx, hist, pbuf, obuf, part):
    c = lax.axis_index("c")
    s = lax.axis_index("s")

    @pl.loop(0, NPAD, step=16)
    def _(r):
        hist[pl.ds(r, 16)] = jnp.zeros((16,), jnp.float32)

    one = jnp.full((16,), 1.0, jnp.float32)

    @pl.loop(0, NSEC)
    def _(k):
        pltpu.sync_copy(edge_hbm.at[1 - c, s, k], sidx)

        @pl.loop(0, SEC)
        def _(i):
            @pl.loop(0, CHUNK, step=16)
            def _(j):
                plsc.addupdate_scatter(hist, [sidx[i, pl.ds(j, 16)]], one)

    pltpu.sync_copy(hist, part.at[s])
    plsc.subcore_barrier()

    base = s * ACC_ROWS_PER_TILE

    @pl.loop(0, NS)
    def _(t):
        pltpu.sync_copy(part.at[t, pl.ds(base, ACC_ROWS_PER_TILE)], pbuf.at[t])

    @pl.loop(0, RED)
    def _(m):
        def body(t, acc):
            return acc + pbuf[t, pl.ds(m * 16, 16)]

        accv = lax.fori_loop(1, NS, body, pbuf[0, pl.ds(m * 16, 16)])
        obuf[m // 8, pl.ds((m % 8) * 16, 16)] = accv

    pltpu.sync_copy(obuf, c_hbm.at[c, s])


BN = 2000


def _tc0_body(x_ref, w1_ref, wr_ref, br_ref, wf0_ref, bf_ref,
              xt_ref, yr_ref, p_ref):
    xb = x_ref[...]
    xt_ref[...] = _dotT(xb, w1_ref[...])
    yr_ref[...] = _dotT(xb, wr_ref[...]) + br_ref[...]
    p_ref[...] = _dotT(xb, wf0_ref[...]) + bf_ref[...]


def _tc0(x, w1, wr, br, wf0, bf):
    return pl.pallas_call(
        _tc0_body,
        grid=(N // BN,),
        in_specs=[
            pl.BlockSpec((BN, D), lambda i: (i, 0)),
            pl.BlockSpec((D, D), lambda i: (0, 0)),
            pl.BlockSpec((D, D), lambda i: (0, 0)),
            pl.BlockSpec((1, D), lambda i: (0, 0)),
            pl.BlockSpec((D, D), lambda i: (0, 0)),
            pl.BlockSpec((1, D), lambda i: (0, 0)),
        ],
        out_specs=[pl.BlockSpec((BN, D), lambda i: (i, 0))] * 3,
        out_shape=[jax.ShapeDtypeStruct((N, D), jnp.float32)] * 3,
    )(x, w1, wr, br, wf0, bf)


def _tcstep_body(yr_ref, s1_ref, s2_ref, c1_ref, c2_ref, p_ref, g_ref, b_ref,
                 wfh_ref, w1_ref, wr_ref, br_ref, xt_ref, yrn_ref, pn_ref):
    r1 = 1.0 / jnp.maximum(c1_ref[...], 1.0)
    r2 = 1.0 / jnp.maximum(c2_ref[...], 1.0)
    t = yr_ref[...] + s1_ref[...] * r1 + s2_ref[...] * r2
    t = jnp.maximum(t, 0.0)
    mu = jnp.mean(t, axis=1, keepdims=True)
    var = jnp.mean((t - mu) * (t - mu), axis=1, keepdims=True)
    h = (t - mu) * jax.lax.rsqrt(var + 1e-5) * g_ref[...] + b_ref[...]
    xt_ref[...] = _dotT(h, w1_ref[...])
    yrn_ref[...] = _dotT(h, wr_ref[...]) + br_ref[...]
    pn_ref[...] = p_ref[...] + _dotT(h, wfh_ref[...])


def _tcstep(yr, s1, s2, c1, c2, p, g, b, wfh, w1n, wrn, brn):
    return pl.pallas_call(
        _tcstep_body,
        grid=(N // BN,),
        in_specs=[
            pl.BlockSpec((BN, D), lambda i: (i, 0)),
            pl.BlockSpec((BN, D), lambda i: (i, 0)),
            pl.BlockSpec((BN, D), lambda i: (i, 0)),
            pl.BlockSpec((BN, 1), lambda i: (i, 0)),
            pl.BlockSpec((BN, 1), lambda i: (i, 0)),
            pl.BlockSpec((BN, D), lambda i: (i, 0)),
            pl.BlockSpec((1, D), lambda i: (0, 0)),
            pl.BlockSpec((1, D), lambda i: (0, 0)),
            pl.BlockSpec((D, D), lambda i: (0, 0)),
            pl.BlockSpec((D, D), lambda i: (0, 0)),
            pl.BlockSpec((D, D), lambda i: (0, 0)),
            pl.BlockSpec((1, D), lambda i: (0, 0)),
        ],
        out_specs=[pl.BlockSpec((BN, D), lambda i: (i, 0))] * 3,
        out_shape=[jax.ShapeDtypeStruct((N, D), jnp.float32)] * 3,
    )(yr, s1, s2, c1, c2, p, g, b, wfh, w1n, wrn, brn)


def kernel(x, edge_index, W1_0, Wr_0, br_0, g_0, b_0, W1_1, Wr_1, br_1, g_1,
           b_1, W1_2, Wr_2, br_2, g_2, b_2, Wf, bf):
    edge_r = edge_index.reshape(2, NS, NSEC, SEC, CHUNK)
    cnt = _counts(edge_r).reshape(2, NPAD)
    c1 = cnt[0, :N].reshape(N, 1)
    c2 = cnt[1, :N].reshape(N, 1)

    xt0, yr0, p0 = _tc0(x, W1_0, Wr_0, br_0.reshape(1, D), Wf[:, :D],
                        bf.reshape(1, D))

    gs = jnp.stack([g_0, g_1, g_2]).reshape(3, 1, D)
    bs = jnp.stack([b_0, b_1, b_2]).reshape(3, 1, D)
    wfh = jnp.stack([Wf[:, D:2 * D], Wf[:, 2 * D:3 * D], Wf[:, 3 * D:4 * D]])
    w1n = jnp.stack([W1_1, W1_2, W1_0])
    wrn = jnp.stack([Wr_1, Wr_2, Wr_0])
    brn = jnp.stack([br_1, br_2, br_0]).reshape(3, 1, D)

    def step(carry, pv):
        xt, yr, p = carry
        g, b, wfh_i, w1_i, wr_i, br_i = pv
        S = _segsum(xt, edge_r)
        xt2, yr2, p2 = _tcstep(yr, S[0, :N], S[1, :N], c1, c2, p, g, b,
                               wfh_i, w1_i, wr_i, br_i)
        return (xt2, yr2, p2), None

    (_, _, p3), _ = jax.lax.scan(step, (xt0, yr0, p0),
                                 (gs, bs, wfh, w1n, wrn, brn))
    return p3


# speedup vs baseline: 9.1574x; 1.0232x over previous
import dataclasses
import functools

import jax
import jax.numpy as jnp
from jax import lax
from jax.experimental import pallas as pl
from jax.experimental.pallas import tpu as pltpu
from jax.experimental.pallas import tpu_sc as plsc

N = 10000
E = 320000
D = 128

NC = 2
NS = 16
CHUNK = 80
EDGES_PER_TILE = E // NS
NCHUNK = EDGES_PER_TILE // CHUNK
SEC = 50
NSEC = NCHUNK // SEC
NPAD = 10240
ACC_ROWS_PER_TILE = NPAD // NS
WB = CHUNK

_mesh = plsc.VectorSubcoreMesh(core_axis_name="c", subcore_axis_name="s")

_sc_params = pltpu.CompilerParams()
if "needs_layout_passes" in pltpu.CompilerParams.__dataclass_fields__:
    _sc_params = dataclasses.replace(_sc_params, needs_layout_passes=False)


def _dotT(a, b):
    return jax.lax.dot_general(
        a, b, (((1,), (1,)), ((), ())),
        precision=jax.lax.Precision.HIGHEST,
        preferred_element_type=jnp.float32)


@functools.partial(
    pl.kernel,
    mesh=_mesh,
    out_type=jax.ShapeDtypeStruct((2, NPAD, D), jnp.float32),
    scratch_types=[
        pltpu.VMEM((SEC, CHUNK), jnp.int32),
        pltpu.VMEM((SEC, CHUNK), jnp.int32),
        pltpu.VMEM((SEC, CHUNK), jnp.int32),
        pltpu.VMEM((SEC, CHUNK), jnp.int32),
        pltpu.VMEM((CHUNK, D), jnp.float32),
        pltpu.VMEM((CHUNK, D), jnp.float32),
        pltpu.VMEM_SHARED((NPAD, D), jnp.float32),
        pltpu.SemaphoreType.DMA,
        pltpu.SemaphoreType.DMA,
        pltpu.SemaphoreType.DMA,
    ])
def _segsum(xt_hbm, edge_hbm, s_hbm, gidx0, sidx0, gidx1, sidx1, rows0, rows1,
            acc, sem0, sem1, isem):
    c = lax.axis_index("c")
    s = lax.axis_index("s")
    base = s * ACC_ROWS_PER_TILE

    def _sec_start(k, gb, sb):
        pltpu.make_async_copy(edge_hbm.at[c, s, k], gb, isem).start()
        pltpu.make_async_copy(edge_hbm.at[1 - c, s, k], sb, isem).start()

    def _sec_wait(gb, sb):
        pltpu.make_async_copy(edge_hbm.at[c, s, 0], gb, isem).wait()
        pltpu.make_async_copy(edge_hbm.at[c, s, 0], sb, isem).wait()

    @pl.loop(0, WB)
    def _(r):
        @pl.loop(0, D, step=16)
        def _(j):
            rows0[r, pl.ds(j, 16)] = jnp.zeros((16,), jnp.float32)

    _sec_start(0, gidx0, sidx0)
    for k in range(ACC_ROWS_PER_TILE // WB):
        pltpu.async_copy(rows0, acc.at[pl.ds(base + k * WB, WB)], sem0)
    for k in range(ACC_ROWS_PER_TILE // WB):
        pltpu.make_async_copy(rows0, acc.at[pl.ds(base, WB)], sem0).wait()

    plsc.subcore_barrier()
    _sec_wait(gidx0, sidx0)

    def _process(gidx, sidx):
        def _g_start(i, buf, sem):
            pltpu.make_async_copy(xt_hbm.at[gidx.at[i]], buf, sem).start()

        def _g_wait(buf, sem):
            pltpu.make_async_copy(xt_hbm.at[gidx.at[0]], buf, sem).wait()

        _g_start(0, rows0, sem0)

        @pl.loop(0, SEC // 2 - 1)
        def _(i2):
            a = 2 * i2
            _g_start(a + 1, rows1, sem1)
            _g_wait(rows0, sem0)
            pltpu.sync_copy(rows0, acc.at[sidx.at[a]], add=True)
            _g_start(a + 2, rows0, sem0)
            _g_wait(rows1, sem1)
            pltpu.sync_copy(rows1, acc.at[sidx.at[a + 1]], add=True)

        _g_start(SEC - 1, rows1, sem1)
        _g_wait(rows0, sem0)
        pltpu.sync_copy(rows0, acc.at[sidx.at[SEC - 2]], add=True)
        _g_wait(rows1, sem1)
        pltpu.sync_copy(rows1, acc.at[sidx.at[SEC - 1]], add=True)

    @pl.loop(0, (NSEC - 1) // 2)
    def _(k2):
        a = 2 * k2
        _sec_start(a + 1, gidx1, sidx1)
        _process(gidx0, sidx0)
        _sec_wait(gidx1, sidx1)
        _sec_start(a + 2, gidx0, sidx0)
        _process(gidx1, sidx1)
        _sec_wait(gidx0, sidx0)

    _process(gidx0, sidx0)

    plsc.subcore_barrier()

    pltpu.sync_copy(acc.at[pl.ds(base, ACC_ROWS_PER_TILE)],
                    s_hbm.at[c, pl.ds(base, ACC_ROWS_PER_TILE)])


RED = ACC_ROWS_PER_TILE // 16


@functools.partial(
    pl.kernel,
    mesh=_mesh,
    out_type=jax.ShapeDtypeStruct((2, NS, ACC_ROWS_PER_TILE // 128, 128),
                                  jnp.float32),
    compiler_params=_sc_params,
    scratch_types=[
        pltpu.VMEM((SEC, CHUNK), jnp.int32),
        pltpu.VMEM((NPAD,), jnp.float32),
        pltpu.VMEM((NS, ACC_ROWS_PER_TILE), jnp.float32),
        pltpu.VMEM((ACC_ROWS_PER_TILE // 128, 128), jnp.float32),
        pltpu.VMEM_SHARED((NS, NPAD), jnp.float32),
    ])
def _counts(edge_hbm, c_hbm, sidx, hist, pbuf, obuf, part):
    c = lax.axis_index("c")
    s = lax.axis_index("s")

    @pl.loop(0, NPAD, step=16)
    def _(r):
        hist[pl.ds(r, 16)] = jnp.zeros((16,), jnp.float32)

    one = jnp.full((16,), 1.0, jnp.float32)

    @pl.loop(0, NSEC)
    def _(k):
        pltpu.sync_copy(edge_hbm.at[1 - c, s, k], sidx)

        @pl.loop(0, SEC)
        def _(i):
            @pl.loop(0, CHUNK, step=16)
            def _(j):
                plsc.addupdate_scatter(hist, [sidx[i, pl.ds(j, 16)]], one)

    pltpu.sync_copy(hist, part.at[s])
    plsc.subcore_barrier()

    base = s * ACC_ROWS_PER_TILE

    @pl.loop(0, NS)
    def _(t):
        pltpu.sync_copy(part.at[t, pl.ds(base, ACC_ROWS_PER_TILE)], pbuf.at[t])

    @pl.loop(0, RED)
    def _(m):
        def body(t, acc):
            return acc + pbuf[t, pl.ds(m * 16, 16)]

        accv = lax.fori_loop(1, NS, body, pbuf[0, pl.ds(m * 16, 16)])
        obuf[m // 8, pl.ds((m % 8) * 16, 16)] = accv

    pltpu.sync_copy(obuf, c_hbm.at[c, s])


BN = 2000


def _tc0_body(x_ref, w1_ref, wr_ref, br_ref, wf0_ref, bf_ref,
              xt_ref, yr_ref, p_ref):
    xb = x_ref[...]
    xt_ref[...] = _dotT(xb, w1_ref[...])
    yr_ref[...] = _dotT(xb, wr_ref[...]) + br_ref[...]
    p_ref[...] = _dotT(xb, wf0_ref[...]) + bf_ref[...]


def _tc0(x, w1, wr, br, wf0, bf):
    return pl.pallas_call(
        _tc0_body,
        grid=(N // BN,),
        in_specs=[
            pl.BlockSpec((BN, D), lambda i: (i, 0)),
            pl.BlockSpec((D, D), lambda i: (0, 0)),
            pl.BlockSpec((D, D), lambda i: (0, 0)),
            pl.BlockSpec((1, D), lambda i: (0, 0)),
            pl.BlockSpec((D, D), lambda i: (0, 0)),
            pl.BlockSpec((1, D), lambda i: (0, 0)),
        ],
        out_specs=[pl.BlockSpec((BN, D), lambda i: (i, 0))] * 3,
        out_shape=[jax.ShapeDtypeStruct((N, D), jnp.float32)] * 3,
    )(x, w1, wr, br, wf0, bf)


def _tcstep_body(yr_ref, s1_ref, s2_ref, c1_ref, c2_ref, p_ref, g_ref, b_ref,
                 wfh_ref, w1_ref, wr_ref, br_ref, xt_ref, yrn_ref, pn_ref):
    r1 = 1.0 / jnp.maximum(c1_ref[...], 1.0)
    r2 = 1.0 / jnp.maximum(c2_ref[...], 1.0)
    t = yr_ref[...] + s1_ref[...] * r1 + s2_ref[...] * r2
    t = jnp.maximum(t, 0.0)
    mu = jnp.mean(t, axis=1, keepdims=True)
    var = jnp.mean((t - mu) * (t - mu), axis=1, keepdims=True)
    h = (t - mu) * jax.lax.rsqrt(var + 1e-5) * g_ref[...] + b_ref[...]
    xt_ref[...] = _dotT(h, w1_ref[...])
    yrn_ref[...] = _dotT(h, wr_ref[...]) + br_ref[...]
    pn_ref[...] = p_ref[...] + _dotT(h, wfh_ref[...])


def _tcstep(yr, s1, s2, c1, c2, p, g, b, wfh, w1n, wrn, brn):
    return pl.pallas_call(
        _tcstep_body,
        grid=(N // BN,),
        in_specs=[
            pl.BlockSpec((BN, D), lambda i: (i, 0)),
            pl.BlockSpec((BN, D), lambda i: (i, 0)),
            pl.BlockSpec((BN, D), lambda i: (i, 0)),
            pl.BlockSpec((BN, 1), lambda i: (i, 0)),
            pl.BlockSpec((BN, 1), lambda i: (i, 0)),
            pl.BlockSpec((BN, D), lambda i: (i, 0)),
            pl.BlockSpec((1, D), lambda i: (0, 0)),
            pl.BlockSpec((1, D), lambda i: (0, 0)),
            pl.BlockSpec((D, D), lambda i: (0, 0)),
            pl.BlockSpec((D, D), lambda i: (0, 0)),
            pl.BlockSpec((D, D), lambda i: (0, 0)),
            pl.BlockSpec((1, D), lambda i: (0, 0)),
        ],
        out_specs=[pl.BlockSpec((BN, D), lambda i: (i, 0))] * 3,
        out_shape=[jax.ShapeDtypeStruct((N, D), jnp.float32)] * 3,
    )(yr, s1, s2, c1, c2, p, g, b, wfh, w1n, wrn, brn)


def kernel(x, edge_index, W1_0, Wr_0, br_0, g_0, b_0, W1_1, Wr_1, br_1, g_1,
           b_1, W1_2, Wr_2, br_2, g_2, b_2, Wf, bf):
    edge_r = edge_index.reshape(2, NS, NSEC, SEC, CHUNK)
    cnt = _counts(edge_r).reshape(2, NPAD)
    c1 = cnt[0, :N].reshape(N, 1)
    c2 = cnt[1, :N].reshape(N, 1)

    xt0, yr0, p0 = _tc0(x, W1_0, Wr_0, br_0.reshape(1, D), Wf[:, :D],
                        bf.reshape(1, D))

    gs = jnp.stack([g_0, g_1, g_2]).reshape(3, 1, D)
    bs = jnp.stack([b_0, b_1, b_2]).reshape(3, 1, D)
    wfh = jnp.stack([Wf[:, D:2 * D], Wf[:, 2 * D:3 * D], Wf[:, 3 * D:4 * D]])
    w1n = jnp.stack([W1_1, W1_2, W1_0])
    wrn = jnp.stack([Wr_1, Wr_2, Wr_0])
    brn = jnp.stack([br_1, br_2, br_0]).reshape(3, 1, D)

    def step(carry, pv):
        xt, yr, p = carry
        g, b, wfh_i, w1_i, wr_i, br_i = pv
        S = _segsum(xt, edge_r)
        xt2, yr2, p2 = _tcstep(yr, S[0, :N], S[1, :N], c1, c2, p, g, b,
                               wfh_i, w1_i, wr_i, br_i)
        return (xt2, yr2, p2), None

    (_, _, p3), _ = jax.lax.scan(step, (xt0, yr0, p0),
                                 (gs, bs, wfh, w1n, wrn, brn))
    return p3
